# baseline (speedup 1.0000x reference)
import functools

import jax
import jax.numpy as jnp
from jax import lax
from jax.experimental import pallas as pl
from jax.experimental.pallas import tpu as pltpu
from jax.experimental.pallas import tpu_sc as plsc

N_SERIES = 64
BATCH = 16384
CTX = 200

NW = 32
RW = BATCH // NW
CHUNK = 64
CHUNK_ELEMS = CHUNK * CTX
N_CHUNKS = RW // CHUNK
PAIRS = CHUNK // 2
VREGS_PER_PAIR = (2 * CTX) // 16


def _body(x_hbm, i_hbm, loc_hbm, scale_hbm, out_hbm,
          xb, ob, loc_v, inv_v, idx_v, a_v, b_v):
    nc = 2
    wid = lax.axis_index("s") * nc + lax.axis_index("c")
    base_row = wid * RW

    pltpu.sync_copy(loc_hbm, loc_v)
    pltpu.sync_copy(scale_hbm, inv_v)
    pltpu.sync_copy(i_hbm.at[pl.ds(base_row, RW)], idx_v)

    one = jnp.full((16,), 1.0, jnp.float32)
    for k in range(N_SERIES // 16):
        sl = pl.ds(16 * k, 16)
        inv_v[sl] = one / inv_v[sl]

    for k in range(RW // 16):
        sl = pl.ds(16 * k, 16)
        iv = idx_v[sl]
        lv = plsc.load_gather(loc_v, [iv])
        nv = plsc.load_gather(inv_v, [iv])
        a_v[sl] = nv
        b_v[sl] = -lv * nv

    lane = lax.iota(jnp.int32, 16)
    mid_mask = lane < 8

    def chunk_body(c, _):
        elem_off = (base_row + c * CHUNK) * CTX
        pltpu.sync_copy(x_hbm.at[pl.ds(elem_off, CHUNK_ELEMS)], xb)

        def pair_body(p, _):
            r0 = c * CHUNK + 2 * p
            i0 = jnp.full((16,), r0, jnp.int32)
            i1 = i0 + 1
            a0 = plsc.load_gather(a_v, [i0])
            b0 = plsc.load_gather(b_v, [i0])
            a1 = plsc.load_gather(a_v, [i1])
            b1 = plsc.load_gather(b_v, [i1])
            am = jnp.where(mid_mask, a0, a1)
            bm = jnp.where(mid_mask, b0, b1)
            pbase = p * (2 * CTX)
            for j in range(VREGS_PER_PAIR):
                if j < 12:
                    a, b = a0, b0
                elif j > 12:
                    a, b = a1, b1
                else:
                    a, b = am, bm
                sl = pl.ds(pbase + 16 * j, 16)
                ob[sl] = xb[sl] * a + b
            return 0

        lax.fori_loop(0, PAIRS, pair_body, 0)
        pltpu.sync_copy(ob, out_hbm.at[pl.ds(elem_off, CHUNK_ELEMS)])
        return 0

    lax.fori_loop(0, N_CHUNKS, chunk_body, 0)


@jax.jit
def kernel(x, i, loc, scale):
    mesh = plsc.VectorSubcoreMesh(
        core_axis_name="c", subcore_axis_name="s", num_cores=2, num_subcores=16
    )
    k = pl.kernel(
        _body,
        out_type=jax.ShapeDtypeStruct((BATCH * CTX,), jnp.float32),
        mesh=mesh,
        compiler_params=pltpu.CompilerParams(needs_layout_passes=False),
        scratch_types=[
            pltpu.VMEM((CHUNK_ELEMS,), jnp.float32),
            pltpu.VMEM((CHUNK_ELEMS,), jnp.float32),
            pltpu.VMEM((N_SERIES,), jnp.float32),
            pltpu.VMEM((N_SERIES,), jnp.float32),
            pltpu.VMEM((RW,), jnp.int32),
            pltpu.VMEM((RW,), jnp.float32),
            pltpu.VMEM((RW,), jnp.float32),
        ],
    )
    out_flat = k(x.reshape(-1), i.astype(jnp.int32), loc, scale)
    return out_flat.reshape(BATCH, CTX)
